# baseline (speedup 1.0000x reference)
import jax
import jax.numpy as jnp
from jax.experimental import pallas as pl
from jax.experimental.pallas import tpu as pltpu

_NC = 8
_C = 2048
_H = _C // 2
_N = 8192
_BLK = 256
_GRID = _N // _BLK


def _bce(z, y):
    return jnp.maximum(z, 0.0) - z * y + jnp.log1p(jnp.exp(-jnp.abs(z)))


def _fused_kernel(x_ref, labels_ref, w1_ref, b1_ref, w2_ref, b2_ref,
                  cp_ref, pn_ref, cls_ref,
                  acc_pn, acc_cls, acc_cnt):
    i = pl.program_id(0)

    @pl.when(i == 0)
    def _init():
        acc_pn[:, :] = jnp.zeros_like(acc_pn)
        acc_cls[:, :] = jnp.zeros_like(acc_cls)
        acc_cnt[:, :] = jnp.zeros_like(acc_cnt)

    xb = x_ref[:, :].astype(jnp.bfloat16)

    cols = []
    mask = None
    maskf = None
    cls_part = jnp.zeros((1, 1), jnp.float32)
    for c in range(_NC):
        h = jax.lax.dot_general(xb, w1_ref[c], (((1,), (1,)), ((), ())),
                                preferred_element_type=jnp.float32)
        h = jnp.maximum(h + b1_ref[c], 0.0)
        hb = h.astype(jnp.bfloat16)
        z = jax.lax.dot_general(hb, w2_ref[c], (((1,), (0,)), ((), ())),
                                preferred_element_type=jnp.float32)
        z = z + b2_ref[c, 0]
        y = labels_ref[:, c:c + 1]
        if c == 0:
            acc_pn[:, :] += jnp.sum(_bce(z, y), axis=(0, 1), keepdims=True)
            probs = jax.nn.sigmoid(z)
            mask = probs >= 0.5
            maskf = mask.astype(jnp.float32)
            acc_cnt[:, :] += jnp.sum(maskf, axis=(0, 1), keepdims=True)
            cols.append(probs)
        else:
            cls_part = cls_part + jnp.sum(_bce(z, y) * maskf,
                                          axis=(0, 1), keepdims=True)
            cols.append(jnp.where(mask, jax.nn.sigmoid(z), 0.0))
    acc_cls[:, :] += cls_part

    cp_ref[:, :] = jnp.concatenate(cols, axis=1)

    @pl.when(i == _GRID - 1)
    def _fin():
        pn_ref[:, :] = acc_pn[:, :] * (1.0 / _N)
        cls_ref[:, :] = acc_cls[:, :] / jnp.maximum(acc_cnt[0, 0], 1.0)


def kernel(fuse_embeddings, segment_classes, pn_W1, pn_b1, pn_W2, pn_b2,
           cat_W1, cat_b1, cat_W2, cat_b2):
    seg = segment_classes.reshape(-1).astype(jnp.int32)
    lab0 = (seg > 0).astype(jnp.float32)
    labc = (seg[:, None] == jnp.arange(1, _NC, dtype=jnp.int32)[None, :]).astype(jnp.float32)
    labels = jnp.concatenate([lab0[:, None], labc], axis=1)

    x = fuse_embeddings.reshape(_N, _C)
    w1 = jnp.concatenate([pn_W1[None].astype(jnp.bfloat16),
                          cat_W1.astype(jnp.bfloat16)], axis=0)
    b1 = jnp.concatenate([pn_b1[None], cat_b1], axis=0).reshape(_NC, 1, _H)
    w2 = jnp.concatenate([pn_W2[None].astype(jnp.bfloat16),
                          cat_W2.astype(jnp.bfloat16)], axis=0).reshape(_NC, _H, 1)
    b2 = jnp.concatenate([pn_b2, cat_b2], axis=0).reshape(_NC, 1)

    out_shapes = (
        jax.ShapeDtypeStruct((_N, _NC), jnp.float32),
        jax.ShapeDtypeStruct((1, 1), jnp.float32),
        jax.ShapeDtypeStruct((1, 1), jnp.float32),
    )
    full = lambda *shape: pl.BlockSpec(shape, lambda i: (0,) * len(shape))
    class_pred, pn, cls = pl.pallas_call(
        _fused_kernel,
        grid=(_GRID,),
        in_specs=[
            pl.BlockSpec((_BLK, _C), lambda i: (i, 0)),
            pl.BlockSpec((_BLK, _NC), lambda i: (i, 0)),
            full(_NC, _H, _C),
            full(_NC, 1, _H),
            full(_NC, _H, 1),
            pl.BlockSpec(memory_space=pltpu.SMEM),
        ],
        out_specs=[
            pl.BlockSpec((_BLK, _NC), lambda i: (i, 0)),
            pl.BlockSpec((1, 1), lambda i: (0, 0)),
            pl.BlockSpec((1, 1), lambda i: (0, 0)),
        ],
        out_shape=out_shapes,
        scratch_shapes=[
            pltpu.VMEM((1, 1), jnp.float32),
            pltpu.VMEM((1, 1), jnp.float32),
            pltpu.VMEM((1, 1), jnp.float32),
        ],
        compiler_params=pltpu.CompilerParams(
            dimension_semantics=("arbitrary",),
            vmem_limit_bytes=64 * 1024 * 1024,
        ),
    )(x, labels, w1, b1, w2, b2)

    return pn[0, 0], cls.reshape(1), class_pred


# speedup vs baseline: 1.0570x; 1.0570x over previous
import jax
import jax.numpy as jnp
from jax.experimental import pallas as pl
from jax.experimental.pallas import tpu as pltpu

_NC = 8
_C = 2048
_H = _C // 2
_N = 8192
_BLK = 256
_GRID = _N // _BLK


def _bce(z, y):
    return jnp.maximum(z, 0.0) - z * y + jnp.log1p(jnp.exp(-jnp.abs(z)))


def _fused_kernel(x_ref, labels_ref, w1_ref, b1_ref, w2_ref, b2_ref,
                  cp_ref, pn_ref, cls_ref,
                  acc_pn, acc_cls, acc_cnt):
    i = pl.program_id(0)

    @pl.when(i == 0)
    def _init():
        acc_pn[:, :] = jnp.zeros_like(acc_pn)
        acc_cls[:, :] = jnp.zeros_like(acc_cls)
        acc_cnt[:, :] = jnp.zeros_like(acc_cnt)

    xb = x_ref[:, :].astype(jnp.bfloat16)

    hbs = []
    for c in range(_NC):
        h = jax.lax.dot_general(xb, w1_ref[c], (((1,), (1,)), ((), ())),
                                preferred_element_type=jnp.float32)
        h = jnp.maximum(h + b1_ref[c], 0.0)
        hbs.append(h.astype(jnp.bfloat16))

    cols = []
    mask = None
    maskf = None
    cls_part = jnp.zeros((1, 1), jnp.float32)
    for c in range(_NC):
        z = jax.lax.dot_general(hbs[c], w2_ref[c], (((1,), (0,)), ((), ())),
                                preferred_element_type=jnp.float32)
        z = z + b2_ref[c, 0]
        y = labels_ref[:, c:c + 1]
        if c == 0:
            acc_pn[:, :] += jnp.sum(_bce(z, y), axis=(0, 1), keepdims=True)
            probs = jax.nn.sigmoid(z)
            mask = probs >= 0.5
            maskf = mask.astype(jnp.float32)
            acc_cnt[:, :] += jnp.sum(maskf, axis=(0, 1), keepdims=True)
            cols.append(probs)
        else:
            cls_part = cls_part + jnp.sum(_bce(z, y) * maskf,
                                          axis=(0, 1), keepdims=True)
            cols.append(jnp.where(mask, jax.nn.sigmoid(z), 0.0))
    acc_cls[:, :] += cls_part

    cp_ref[:, :] = jnp.concatenate(cols, axis=1)

    @pl.when(i == _GRID - 1)
    def _fin():
        pn_ref[:, :] = acc_pn[:, :] * (1.0 / _N)
        cls_ref[:, :] = acc_cls[:, :] / jnp.maximum(acc_cnt[0, 0], 1.0)


def kernel(fuse_embeddings, segment_classes, pn_W1, pn_b1, pn_W2, pn_b2,
           cat_W1, cat_b1, cat_W2, cat_b2):
    seg = segment_classes.reshape(-1).astype(jnp.int32)
    lab0 = (seg > 0).astype(jnp.float32)
    labc = (seg[:, None] == jnp.arange(1, _NC, dtype=jnp.int32)[None, :]).astype(jnp.float32)
    labels = jnp.concatenate([lab0[:, None], labc], axis=1)

    x = fuse_embeddings.reshape(_N, _C)
    w1 = jnp.concatenate([pn_W1[None].astype(jnp.bfloat16),
                          cat_W1.astype(jnp.bfloat16)], axis=0)
    b1 = jnp.concatenate([pn_b1[None], cat_b1], axis=0).reshape(_NC, 1, _H)
    w2 = jnp.concatenate([pn_W2[None].astype(jnp.bfloat16),
                          cat_W2.astype(jnp.bfloat16)], axis=0).reshape(_NC, _H, 1)
    b2 = jnp.concatenate([pn_b2, cat_b2], axis=0).reshape(_NC, 1)

    out_shapes = (
        jax.ShapeDtypeStruct((_N, _NC), jnp.float32),
        jax.ShapeDtypeStruct((1, 1), jnp.float32),
        jax.ShapeDtypeStruct((1, 1), jnp.float32),
    )
    full = lambda *shape: pl.BlockSpec(shape, lambda i: (0,) * len(shape))
    class_pred, pn, cls = pl.pallas_call(
        _fused_kernel,
        grid=(_GRID,),
        in_specs=[
            pl.BlockSpec((_BLK, _C), lambda i: (i, 0)),
            pl.BlockSpec((_BLK, _NC), lambda i: (i, 0)),
            full(_NC, _H, _C),
            full(_NC, 1, _H),
            full(_NC, _H, 1),
            pl.BlockSpec(memory_space=pltpu.SMEM),
        ],
        out_specs=[
            pl.BlockSpec((_BLK, _NC), lambda i: (i, 0)),
            pl.BlockSpec((1, 1), lambda i: (0, 0)),
            pl.BlockSpec((1, 1), lambda i: (0, 0)),
        ],
        out_shape=out_shapes,
        scratch_shapes=[
            pltpu.VMEM((1, 1), jnp.float32),
            pltpu.VMEM((1, 1), jnp.float32),
            pltpu.VMEM((1, 1), jnp.float32),
        ],
        compiler_params=pltpu.CompilerParams(
            dimension_semantics=("arbitrary",),
            vmem_limit_bytes=64 * 1024 * 1024,
        ),
    )(x, labels, w1, b1, w2, b2)

    return pn[0, 0], cls.reshape(1), class_pred
